# baseline (speedup 1.0000x reference)
import functools

import jax
import jax.numpy as jnp
from jax.experimental import pallas as pl
from jax.experimental.pallas import tpu as pltpu


def _stream_kernel(b_ref, x_hbm, o_ref, c_ref, bufs, sems, *,
                   num_graphs, chunk_n, depth, chunks_per_core):
    ci = pl.program_id(0)
    row0 = ci * chunks_per_core * chunk_n

    def _copy(i):
        slot = jax.lax.rem(i, depth)
        return pltpu.make_async_copy(
            x_hbm.at[pl.ds(row0 + i * chunk_n, chunk_n), :],
            bufs.at[slot], sems.at[slot])

    o_ref[...] = jnp.zeros_like(o_ref)
    c_ref[...] = jnp.zeros_like(c_ref)

    for s in range(depth):
        _copy(s).start()

    def body(i, carry):
        slot = jax.lax.rem(i, depth)
        _copy(i).wait()
        b = b_ref[ci * chunks_per_core + i]
        gids = jax.lax.broadcasted_iota(jnp.int32, (num_graphs, chunk_n), 0)
        m = (gids == b).astype(jnp.bfloat16)
        c_ref[...] += jnp.sum(m, axis=1, keepdims=True,
                              dtype=jnp.float32)[None]
        x = bufs[slot].astype(jnp.bfloat16)
        o_ref[...] += jnp.dot(m, x,
                              preferred_element_type=jnp.float32)[None]

        @pl.when(i + depth < chunks_per_core)
        def _next():
            _copy(i + depth).start()

        return carry

    jax.lax.fori_loop(0, chunks_per_core, body, 0)


def _combine_kernel(p_ref, c_ref, o_ref):
    c = c_ref[0] + c_ref[1]
    p = p_ref[0] + p_ref[1]
    o_ref[...] = p / jnp.maximum(c, 1.0)


def _reduce_mean(node_feat, batch, num_graphs, chunk_n=8192, depth=3,
                 tile_f=128):
    n, f = node_feat.shape
    total_chunks = n // chunk_n
    half = total_chunks // 2

    b3 = batch.astype(jnp.int32).reshape(total_chunks, 1, chunk_n)
    partial, cnt = pl.pallas_call(
        functools.partial(_stream_kernel, num_graphs=num_graphs,
                          chunk_n=chunk_n, depth=depth, chunks_per_core=half),
        out_shape=(jax.ShapeDtypeStruct((2, num_graphs, f), jnp.float32),
                   jax.ShapeDtypeStruct((2, num_graphs, 1), jnp.float32)),
        grid=(2,),
        in_specs=[
            pl.BlockSpec((total_chunks, 1, chunk_n), lambda ci: (0, 0, 0)),
            pl.BlockSpec(memory_space=pl.ANY),
        ],
        out_specs=(pl.BlockSpec((1, num_graphs, f), lambda ci: (ci, 0, 0)),
                   pl.BlockSpec((1, num_graphs, 1), lambda ci: (ci, 0, 0))),
        scratch_shapes=[
            pltpu.VMEM((depth, chunk_n, f), jnp.float32),
            pltpu.SemaphoreType.DMA((depth,)),
        ],
        compiler_params=pltpu.CompilerParams(
            dimension_semantics=("parallel",)),
    )(b3, node_feat)

    return pl.pallas_call(
        _combine_kernel,
        out_shape=jax.ShapeDtypeStruct((num_graphs, f), jnp.float32),
        grid=(f // tile_f,),
        in_specs=[
            pl.BlockSpec((2, num_graphs, tile_f), lambda fi: (0, 0, fi)),
            pl.BlockSpec((2, num_graphs, 1), lambda fi: (0, 0, 0)),
        ],
        out_specs=pl.BlockSpec((num_graphs, tile_f), lambda fi: (0, fi)),
        compiler_params=pltpu.CompilerParams(
            dimension_semantics=("parallel",)),
    )(partial, cnt)


def kernel(node_feat, batch):
    return _reduce_mean(jnp.asarray(node_feat), jnp.asarray(batch), 512)


# speedup vs baseline: 1.1078x; 1.1078x over previous
import functools

import jax
import jax.numpy as jnp
from jax.experimental import pallas as pl
from jax.experimental.pallas import tpu as pltpu


def _partial_kernel(b_ref, x_ref, o_ref, c_ref, *, num_graphs):
    ni = pl.program_id(1)

    @pl.when(ni == 0)
    def _init():
        o_ref[...] = jnp.zeros_like(o_ref)
        c_ref[...] = jnp.zeros_like(c_ref)

    b = b_ref[...]
    gids = jax.lax.broadcasted_iota(jnp.int32, (num_graphs, b.shape[1]), 0)
    m = (gids == b).astype(jnp.bfloat16)
    c_ref[...] += jnp.sum(m, axis=1, keepdims=True,
                          dtype=jnp.float32)[None]
    x = x_ref[...].astype(jnp.bfloat16)
    o_ref[...] += jnp.dot(m, x,
                          preferred_element_type=jnp.float32)[None]


def _combine_kernel(p_ref, c_ref, o_ref):
    c = c_ref[0] + c_ref[1]
    p = p_ref[0] + p_ref[1]
    o_ref[...] = p / jnp.maximum(c, 1.0)


def _reduce_mean(node_feat, batch, num_graphs, tile_n=8192, tile_f=128):
    n, f = node_feat.shape
    num_n = n // tile_n
    half = num_n // 2

    b2 = batch.astype(jnp.int32).reshape(1, n)
    partial, cnt = pl.pallas_call(
        functools.partial(_partial_kernel, num_graphs=num_graphs),
        out_shape=(jax.ShapeDtypeStruct((2, num_graphs, f), jnp.float32),
                   jax.ShapeDtypeStruct((2, num_graphs, 1), jnp.float32)),
        grid=(2, half),
        in_specs=[
            pl.BlockSpec((1, tile_n), lambda ci, ni: (0, ci * half + ni)),
            pl.BlockSpec((tile_n, f), lambda ci, ni: (ci * half + ni, 0)),
        ],
        out_specs=(pl.BlockSpec((1, num_graphs, f), lambda ci, ni: (ci, 0, 0)),
                   pl.BlockSpec((1, num_graphs, 1), lambda ci, ni: (ci, 0, 0))),
        compiler_params=pltpu.CompilerParams(
            dimension_semantics=("parallel", "arbitrary")),
    )(b2, node_feat)

    return pl.pallas_call(
        _combine_kernel,
        out_shape=jax.ShapeDtypeStruct((num_graphs, f), jnp.float32),
        grid=(f // tile_f,),
        in_specs=[
            pl.BlockSpec((2, num_graphs, tile_f), lambda fi: (0, 0, fi)),
            pl.BlockSpec((2, num_graphs, 1), lambda fi: (0, 0, 0)),
        ],
        out_specs=pl.BlockSpec((num_graphs, tile_f), lambda fi: (0, fi)),
        compiler_params=pltpu.CompilerParams(
            dimension_semantics=("parallel",)),
    )(partial, cnt)


def kernel(node_feat, batch):
    return _reduce_mean(jnp.asarray(node_feat), jnp.asarray(batch), 512)
